# Initial kernel scaffold; baseline (speedup 1.0000x reference)
import jax
import jax.numpy as jnp
from jax.experimental import pallas as pl


def kernel(key_cache, block_tables, positions):
    raise NotImplementedError("write your pallas kernel here")


# speedup vs baseline: 1.8016x; 1.8016x over previous
import jax
import jax.numpy as jnp
from jax.experimental import pallas as pl
from jax.experimental.pallas import tpu as pltpu

_SINK_SIZE = 16
_CACHE_SIZE = 4096.0 + 16.0
_NUM_KV_HEADS = 8
_HEAD_SIZE = 128
_NUM_BLOCKS = 2048
_BLOCK_SIZE = 16
_BS = 64
_G = 16


def _rotate_body(sb_ref, pos_ref, x_ref, o_ref):
    g = pl.program_id(0)
    bid = jax.lax.broadcasted_iota(jnp.int32, (_G, _BS), 0) + g * _G
    barange = jax.lax.broadcasted_iota(jnp.int32, (_G, _BS), 1)
    match = bid == sb_ref[0:1, :]
    key = jnp.where(match, barange, -1)
    w = jnp.max(key, axis=1, keepdims=True)
    onehot = jnp.logical_and(barange == w, match)
    posf = pos_ref[0:1, :].astype(jnp.float32)
    ev = jnp.maximum(posf - _CACHE_SIZE, 0.0)
    theta = jnp.sum(jnp.where(onehot, ev, 0.0), axis=1, keepdims=True)

    d8i = jax.lax.broadcasted_iota(jnp.int32, (8, 128), 0)
    li = jax.lax.broadcasted_iota(jnp.int32, (8, 128), 1) % 8
    j = (d8i * 8 + li).astype(jnp.float32)
    inv_freq = jnp.exp(j * (-jnp.log(10000.0) / 64.0))
    freqs = theta[:, :, None, None] * inv_freq[None, None, :, :]
    c = jnp.cos(freqs)
    s = jnp.sin(freqs)

    x = x_ref[...]
    x1 = x[:, :, :8, :]
    x2 = x[:, :, 8:, :]
    o_ref[:, :, :8, :] = x1 * c - x2 * s
    o_ref[:, :, 8:, :] = x2 * c + x1 * s


def kernel(key_cache, block_tables, positions):
    n, h, d8, bs, lanes = key_cache.shape
    x = key_cache.reshape(n, h, d8 * bs // 16, 128)
    sb = block_tables[:, :1].reshape(1, _BS)
    pos = positions.reshape(1, _BS)
    out = pl.pallas_call(
        _rotate_body,
        grid=(_NUM_BLOCKS // _G,),
        in_specs=[
            pl.BlockSpec((1, _BS), lambda i: (0, 0)),
            pl.BlockSpec((1, _BS), lambda i: (0, 0)),
            pl.BlockSpec((_G, _NUM_KV_HEADS, 16, 128), lambda i: (i, 0, 0, 0)),
        ],
        out_specs=pl.BlockSpec((_G, _NUM_KV_HEADS, 16, 128), lambda i: (i, 0, 0, 0)),
        out_shape=jax.ShapeDtypeStruct((_NUM_BLOCKS, _NUM_KV_HEADS, 16, 128), jnp.float32),
        compiler_params=pltpu.CompilerParams(
            dimension_semantics=("arbitrary",),
        ),
    )(sb, pos, x)
    return out.reshape(key_cache.shape)


# speedup vs baseline: 2.1065x; 1.1692x over previous
import jax
import jax.numpy as jnp
from jax.experimental import pallas as pl
from jax.experimental.pallas import tpu as pltpu

_SINK_SIZE = 16
_CACHE_SIZE = 4096.0 + 16.0
_NUM_KV_HEADS = 8
_HEAD_SIZE = 128
_NUM_BLOCKS = 2048
_BLOCK_SIZE = 16
_BS = 64
_G = 64


def _rotate_body(sb_ref, pos_ref, x_ref, o_ref):
    g = pl.program_id(0)
    bid = jax.lax.broadcasted_iota(jnp.int32, (_G, _BS), 0) + g * _G
    barange = jax.lax.broadcasted_iota(jnp.int32, (_G, _BS), 1)
    match = bid == sb_ref[0:1, :]
    key = jnp.where(match, barange, -1)
    w = jnp.max(key, axis=1, keepdims=True)
    onehot = jnp.logical_and(barange == w, match)
    posf = pos_ref[0:1, :].astype(jnp.float32)
    ev = jnp.maximum(posf - _CACHE_SIZE, 0.0)
    theta = jnp.sum(jnp.where(onehot, ev, 0.0), axis=1, keepdims=True)

    d8i = jax.lax.broadcasted_iota(jnp.int32, (8, 128), 0)
    li = jax.lax.broadcasted_iota(jnp.int32, (8, 128), 1) % 8
    j = (d8i * 8 + li).astype(jnp.float32)
    inv_freq = jnp.exp(j * (-jnp.log(10000.0) / 64.0))
    freqs = theta[:, :, None, None] * inv_freq[None, None, :, :]
    c = jnp.cos(freqs)
    s = jnp.sin(freqs)

    x = x_ref[...]
    x1 = x[:, :, :8, :]
    x2 = x[:, :, 8:, :]
    o_ref[:, :, :8, :] = x1 * c - x2 * s
    o_ref[:, :, 8:, :] = x2 * c + x1 * s


def kernel(key_cache, block_tables, positions):
    n, h, d8, bs, lanes = key_cache.shape
    x = key_cache.reshape(n, h, d8 * bs // 16, 128)
    sb = block_tables[:, :1].reshape(1, _BS)
    pos = positions.reshape(1, _BS)
    out = pl.pallas_call(
        _rotate_body,
        grid=(_NUM_BLOCKS // _G,),
        in_specs=[
            pl.BlockSpec((1, _BS), lambda i: (0, 0)),
            pl.BlockSpec((1, _BS), lambda i: (0, 0)),
            pl.BlockSpec((_G, _NUM_KV_HEADS, 16, 128), lambda i: (i, 0, 0, 0)),
        ],
        out_specs=pl.BlockSpec((_G, _NUM_KV_HEADS, 16, 128), lambda i: (i, 0, 0, 0)),
        out_shape=jax.ShapeDtypeStruct((_NUM_BLOCKS, _NUM_KV_HEADS, 16, 128), jnp.float32),
        compiler_params=pltpu.CompilerParams(
            dimension_semantics=("arbitrary",),
        ),
    )(sb, pos, x)
    return out.reshape(key_cache.shape)


# speedup vs baseline: 6.9884x; 3.3175x over previous
import jax
import jax.numpy as jnp
from jax.experimental import pallas as pl
from jax.experimental.pallas import tpu as pltpu

_CACHE_SIZE = 4096.0 + 16.0
_NUM_KV_HEADS = 8
_NUM_BLOCKS = 2048
_BS = 64
_C = 128
_GH = 2


def _rotate_body(sb_ref, pos_ref, x_ref, o_ref):
    i = pl.program_id(1)
    bid = jax.lax.broadcasted_iota(jnp.int32, (_BS, _C), 1) + i * _C
    barange = jax.lax.broadcasted_iota(jnp.int32, (_BS, _C), 0)
    match = bid == sb_ref[...]
    key = jnp.where(match, barange, -1)
    w = jnp.max(key, axis=0, keepdims=True)
    onehot = jnp.logical_and(barange == w, match)
    posf = pos_ref[...].astype(jnp.float32)
    ev = jnp.maximum(posf - _CACHE_SIZE, 0.0)
    theta = jnp.sum(jnp.where(onehot, ev, 0.0), axis=0, keepdims=True)

    d8i = jax.lax.broadcasted_iota(jnp.int32, (1, 8, 1, 8, _C), 1)
    li = jax.lax.broadcasted_iota(jnp.int32, (1, 8, 1, 8, _C), 3)
    j = (d8i * 8 + li).astype(jnp.float32)
    inv_freq = jnp.exp(j * (-jnp.log(10000.0) / 64.0))
    freqs = theta.reshape(1, 1, 1, 1, _C) * inv_freq
    c = jnp.cos(freqs)
    s = jnp.sin(freqs)

    x = x_ref[...]
    x1 = x[:, :8]
    x2 = x[:, 8:]
    o_ref[:, :8] = x1 * c - x2 * s
    o_ref[:, 8:] = x2 * c + x1 * s


def kernel(key_cache, block_tables, positions):
    x = jnp.transpose(key_cache, (1, 2, 3, 4, 0))
    sb = block_tables[:, :1]
    pos = positions.reshape(_BS, 1)
    out = pl.pallas_call(
        _rotate_body,
        grid=(_NUM_KV_HEADS // _GH, _NUM_BLOCKS // _C),
        in_specs=[
            pl.BlockSpec((_BS, 1), lambda h, i: (0, 0)),
            pl.BlockSpec((_BS, 1), lambda h, i: (0, 0)),
            pl.BlockSpec((_GH, 16, 16, 8, _C), lambda h, i: (h, 0, 0, 0, i)),
        ],
        out_specs=pl.BlockSpec((_GH, 16, 16, 8, _C), lambda h, i: (h, 0, 0, 0, i)),
        out_shape=jax.ShapeDtypeStruct((_NUM_KV_HEADS, 16, 16, 8, _NUM_BLOCKS), jnp.float32),
        compiler_params=pltpu.CompilerParams(
            dimension_semantics=("arbitrary", "arbitrary"),
        ),
    )(sb, pos, x)
    return jnp.transpose(out, (4, 0, 1, 2, 3))


# speedup vs baseline: 8.2619x; 1.1822x over previous
import jax
import jax.numpy as jnp
from jax.experimental import pallas as pl
from jax.experimental.pallas import tpu as pltpu

_CACHE_SIZE = 4096.0 + 16.0
_NUM_KV_HEADS = 8
_NUM_BLOCKS = 2048
_BS = 64
_C = 128
_GH = 4


def _rotate_body(sb_ref, pos_ref, x_ref, o_ref):
    i = pl.program_id(1)
    bid = jax.lax.broadcasted_iota(jnp.int32, (_BS, _C), 1) + i * _C
    barange = jax.lax.broadcasted_iota(jnp.int32, (_BS, _C), 0)
    match = bid == sb_ref[...]
    key = jnp.where(match, barange, -1)
    w = jnp.max(key, axis=0, keepdims=True)
    onehot = jnp.logical_and(barange == w, match)
    posf = pos_ref[...].astype(jnp.float32)
    ev = jnp.maximum(posf - _CACHE_SIZE, 0.0)
    theta = jnp.sum(jnp.where(onehot, ev, 0.0), axis=0, keepdims=True)

    d8i = jax.lax.broadcasted_iota(jnp.int32, (1, 8, 1, 8, _C), 1)
    li = jax.lax.broadcasted_iota(jnp.int32, (1, 8, 1, 8, _C), 3)
    j = (d8i * 8 + li).astype(jnp.float32)
    inv_freq = jnp.exp(j * (-jnp.log(10000.0) / 64.0))
    freqs = theta.reshape(1, 1, 1, 1, _C) * inv_freq
    c = jnp.cos(freqs)
    s = jnp.sin(freqs)

    x = x_ref[...]
    x1 = x[:, :8]
    x2 = x[:, 8:]
    o_ref[:, :8] = x1 * c - x2 * s
    o_ref[:, 8:] = x2 * c + x1 * s


def kernel(key_cache, block_tables, positions):
    x = jnp.transpose(key_cache, (1, 2, 3, 4, 0))
    sb = block_tables[:, :1]
    pos = positions.reshape(_BS, 1)
    out = pl.pallas_call(
        _rotate_body,
        grid=(_NUM_KV_HEADS // _GH, _NUM_BLOCKS // _C),
        in_specs=[
            pl.BlockSpec((_BS, 1), lambda h, i: (0, 0)),
            pl.BlockSpec((_BS, 1), lambda h, i: (0, 0)),
            pl.BlockSpec((_GH, 16, 16, 8, _C), lambda h, i: (h, 0, 0, 0, i)),
        ],
        out_specs=pl.BlockSpec((_GH, 16, 16, 8, _C), lambda h, i: (h, 0, 0, 0, i)),
        out_shape=jax.ShapeDtypeStruct((_NUM_KV_HEADS, 16, 16, 8, _NUM_BLOCKS), jnp.float32),
        compiler_params=pltpu.CompilerParams(
            dimension_semantics=("arbitrary", "arbitrary"),
        ),
    )(sb, pos, x)
    return jnp.transpose(out, (4, 0, 1, 2, 3))
